# baseline (speedup 1.0000x reference)
import jax
import jax.numpy as jnp
from jax.experimental import pallas as pl
from jax.experimental.pallas import tpu as pltpu

_CH_ROWS = 2048
_NBUF = 3
_PF = 2


def _body(x_hbm, o_hbm, buf, in_sems, out_sems):
    n_rows = x_hbm.shape[0]
    n = n_rows // _CH_ROWS

    def in_copy(i):
        b = i % _NBUF
        return pltpu.make_async_copy(
            x_hbm.at[pl.ds(i * _CH_ROWS, _CH_ROWS), :],
            buf.at[b], in_sems.at[b])

    def out_copy(i):
        b = i % _NBUF
        return pltpu.make_async_copy(
            buf.at[b],
            o_hbm.at[pl.ds(i * _CH_ROWS, _CH_ROWS), :], out_sems.at[b])

    for i in range(_PF):
        in_copy(i).start()

    for i in range(n):
        b = i % _NBUF
        if i + _PF < n:
            if i + _PF >= _NBUF:
                out_copy(i + _PF - _NBUF).wait()
            in_copy(i + _PF).start()
        in_copy(i).wait()
        buf[b] = jnp.clip(buf[b], 0.0, 1.0)
        out_copy(i).start()

    for i in range(n - _NBUF, n):
        out_copy(i).wait()


def kernel(x):
    n_rows, n_cols = x.shape
    return pl.pallas_call(
        _body,
        in_specs=[pl.BlockSpec(memory_space=pl.ANY)],
        out_specs=pl.BlockSpec(memory_space=pl.ANY),
        out_shape=jax.ShapeDtypeStruct((n_rows, n_cols), x.dtype),
        scratch_shapes=[
            pltpu.VMEM((_NBUF, _CH_ROWS, n_cols), x.dtype),
            pltpu.SemaphoreType.DMA((_NBUF,)),
            pltpu.SemaphoreType.DMA((_NBUF,)),
        ],
        compiler_params=pltpu.CompilerParams(
            vmem_limit_bytes=60 * 1024 * 1024,
        ),
    )(x)
